# Initial kernel scaffold; baseline (speedup 1.0000x reference)
import jax
import jax.numpy as jnp
from jax.experimental import pallas as pl


def kernel(x, weight, bias):
    raise NotImplementedError("write your pallas kernel here")


# speedup vs baseline: 1.1049x; 1.1049x over previous
import jax
import jax.numpy as jnp
from jax.experimental import pallas as pl
from jax.experimental.pallas import tpu as pltpu

_EPS = 1e-5
_VMEM_LIMIT = 64 * 1024 * 1024


def _affine_relu(x, w, b):
    y = jnp.dot(w, x.astype(jnp.float32), preferred_element_type=jnp.float32)
    return jnp.maximum(y + b, 0.0)


def _make_stats_kernel(rows_per_step, lanes, t_valid):
    def stats_kernel(x_ref, w_ref, b_ref, sum_ref, sumsq_ref):
        @pl.when(pl.program_id(1) == 0)
        def _init():
            sum_ref[...] = jnp.zeros_like(sum_ref)
            sumsq_ref[...] = jnp.zeros_like(sumsq_ref)

        w = w_ref[...]
        b = b_ref[...]
        h = w.shape[0]
        t = x_ref.shape[-1]
        if t_valid != t:
            col = jax.lax.broadcasted_iota(jnp.int32, (1, t), 1)
            valid = col < t_valid
        s = jnp.zeros((h, lanes), jnp.float32)
        s2 = jnp.zeros((h, lanes), jnp.float32)
        for i in range(rows_per_step):
            y = _affine_relu(x_ref[i], w, b)
            if t_valid != t:
                y = jnp.where(valid, y, 0.0)
            for j in range(t // lanes):
                yj = y[:, j * lanes:(j + 1) * lanes]
                s = s + yj
                s2 = s2 + yj * yj
        sum_ref[...] += s
        sumsq_ref[...] += s2

    return stats_kernel


def _make_norm_kernel(rows_per_step):
    def norm_kernel(x_ref, w_ref, b_ref, m_ref, o_ref):
        w = w_ref[...]
        b = b_ref[...]
        m = m_ref[...]
        for i in range(rows_per_step):
            y = _affine_relu(x_ref[i], w, b)
            o_ref[i] = (y - m).astype(o_ref.dtype)

    return norm_kernel


def kernel(x, weight, bias):
    N, Cin, T = x.shape
    H = weight.shape[0]
    out_dtype = x.dtype
    lanes = 128

    T_pad = (T + lanes - 1) // lanes * lanes
    padded = T_pad != T
    xp = jnp.pad(x, ((0, 0), (0, 0), (0, T_pad - T))) if padded else x

    w2d = weight[:, :, 0].astype(jnp.float32)
    b2d = bias.reshape(H, 1).astype(jnp.float32)

    G = 2 if N % 2 == 0 else 1
    rows = N // G

    stats_x_spec = pl.BlockSpec((1, Cin, T_pad), lambda c, n: (c * rows + n, 0, 0))
    w_spec = pl.BlockSpec((H, Cin), lambda c, n: (0, 0))
    col_spec = pl.BlockSpec((H, 1), lambda c, n: (0, 0))
    acc_spec = pl.BlockSpec((1, H, lanes), lambda c, n: (c, 0, 0))

    sums, sumsqs = pl.pallas_call(
        _make_stats_kernel(1, lanes, T),
        out_shape=(jax.ShapeDtypeStruct((G, H, lanes), jnp.float32),
                   jax.ShapeDtypeStruct((G, H, lanes), jnp.float32)),
        grid=(G, rows),
        in_specs=[stats_x_spec, w_spec, col_spec],
        out_specs=(acc_spec, acc_spec),
        compiler_params=pltpu.CompilerParams(
            dimension_semantics=("parallel", "arbitrary"),
            vmem_limit_bytes=_VMEM_LIMIT),
    )(xp, w2d, b2d)

    count = float(N) * float(T)
    sum_c = jnp.sum(sums, axis=(0, 2)).reshape(H, 1)
    sumsq_c = jnp.sum(sumsqs, axis=(0, 2)).reshape(H, 1)
    mean = sum_c / count
    var = jnp.maximum(sumsq_c / count - mean * mean, 0.0)
    inv = jax.lax.rsqrt(var + _EPS)
    w_s = w2d * inv
    b_s = b2d * inv
    m_s = mean * inv

    Nb = 1
    norm_x_spec = pl.BlockSpec((Nb, Cin, T_pad), lambda n: (n, 0, 0))
    norm_w_spec = pl.BlockSpec((H, Cin), lambda n: (0, 0))
    norm_col_spec = pl.BlockSpec((H, 1), lambda n: (0, 0))
    out_spec = pl.BlockSpec((Nb, H, T_pad), lambda n: (n, 0, 0))

    out = pl.pallas_call(
        _make_norm_kernel(Nb),
        out_shape=jax.ShapeDtypeStruct((N, H, T_pad), out_dtype),
        grid=(N // Nb,),
        in_specs=[norm_x_spec, norm_w_spec, norm_col_spec, norm_col_spec],
        out_specs=out_spec,
        compiler_params=pltpu.CompilerParams(
            dimension_semantics=("parallel",),
            vmem_limit_bytes=_VMEM_LIMIT),
    )(xp, w_s, b_s, m_s)

    return out[:, :, :T] if padded else out


# speedup vs baseline: 1.7626x; 1.5952x over previous
import jax
import jax.numpy as jnp
from jax.experimental import pallas as pl
from jax.experimental.pallas import tpu as pltpu

_EPS = 1e-5
_VMEM_LIMIT = 64 * 1024 * 1024


def _affine_relu(x, w, b):
    y = jnp.dot(w, x.astype(jnp.float32), preferred_element_type=jnp.float32)
    return jnp.maximum(y + b, 0.0)


def _make_stats_kernel(rows_per_step, lanes, t_valid):
    def stats_kernel(x_ref, w_ref, b_ref, sum_ref, sumsq_ref):
        @pl.when(pl.program_id(0) == 0)
        def _init():
            sum_ref[...] = jnp.zeros_like(sum_ref)
            sumsq_ref[...] = jnp.zeros_like(sumsq_ref)

        w = w_ref[...]
        b = b_ref[...]
        h = w.shape[0]
        t = x_ref.shape[-1]
        if t_valid != t:
            col = jax.lax.broadcasted_iota(jnp.int32, (1, t), 1)
            valid = col < t_valid
        s = jnp.zeros((h, lanes), jnp.float32)
        s2 = jnp.zeros((h, lanes), jnp.float32)
        for i in range(rows_per_step):
            y = _affine_relu(x_ref[i], w, b)
            if t_valid != t:
                y = jnp.where(valid, y, 0.0)
            for j in range(t // lanes):
                yj = y[:, j * lanes:(j + 1) * lanes]
                s = s + yj
                s2 = s2 + yj * yj
        sum_ref[...] += s
        sumsq_ref[...] += s2

    return stats_kernel


def _make_norm_kernel(rows_per_step):
    def norm_kernel(x_ref, w_ref, b_ref, m_ref, o_ref):
        w = w_ref[...]
        b = b_ref[...]
        m = m_ref[...]
        for i in range(rows_per_step):
            y = _affine_relu(x_ref[i], w, b)
            o_ref[i] = (y - m).astype(o_ref.dtype)

    return norm_kernel


def kernel(x, weight, bias):
    N, Cin, T = x.shape
    H = weight.shape[0]
    out_dtype = x.dtype
    lanes = 128

    T_pad = (T + lanes - 1) // lanes * lanes
    padded = T_pad != T
    xp = jnp.pad(x, ((0, 0), (0, 0), (0, T_pad - T))) if padded else x

    w2d = weight[:, :, 0].astype(jnp.float32)
    b2d = bias.reshape(H, 1).astype(jnp.float32)

    R = 8
    while N % R:
        R //= 2

    stats_x_spec = pl.BlockSpec((R, Cin, T_pad), lambda n: (n, 0, 0))
    w_spec = pl.BlockSpec((H, Cin), lambda n: (0, 0))
    col_spec = pl.BlockSpec((H, 1), lambda n: (0, 0))
    acc_spec = pl.BlockSpec((1, H, lanes), lambda n: (0, 0, 0))

    sums, sumsqs = pl.pallas_call(
        _make_stats_kernel(R, lanes, T),
        out_shape=(jax.ShapeDtypeStruct((1, H, lanes), jnp.float32),
                   jax.ShapeDtypeStruct((1, H, lanes), jnp.float32)),
        grid=(N // R,),
        in_specs=[stats_x_spec, w_spec, col_spec],
        out_specs=(acc_spec, acc_spec),
        compiler_params=pltpu.CompilerParams(
            dimension_semantics=("arbitrary",),
            vmem_limit_bytes=_VMEM_LIMIT),
    )(xp, w2d, b2d)

    count = float(N) * float(T)
    sum_c = jnp.sum(sums, axis=(0, 2)).reshape(H, 1)
    sumsq_c = jnp.sum(sumsqs, axis=(0, 2)).reshape(H, 1)
    mean = sum_c / count
    var = jnp.maximum(sumsq_c / count - mean * mean, 0.0)
    inv = jax.lax.rsqrt(var + _EPS)
    w_s = w2d * inv
    b_s = b2d * inv
    m_s = mean * inv

    Nb = 4
    while N % Nb:
        Nb //= 2
    norm_x_spec = pl.BlockSpec((Nb, Cin, T_pad), lambda n: (n, 0, 0))
    norm_w_spec = pl.BlockSpec((H, Cin), lambda n: (0, 0))
    norm_col_spec = pl.BlockSpec((H, 1), lambda n: (0, 0))
    out_spec = pl.BlockSpec((Nb, H, T_pad), lambda n: (n, 0, 0))

    out = pl.pallas_call(
        _make_norm_kernel(Nb),
        out_shape=jax.ShapeDtypeStruct((N, H, T_pad), out_dtype),
        grid=(N // Nb,),
        in_specs=[norm_x_spec, norm_w_spec, norm_col_spec, norm_col_spec],
        out_specs=out_spec,
        compiler_params=pltpu.CompilerParams(
            dimension_semantics=("arbitrary",),
            vmem_limit_bytes=_VMEM_LIMIT),
    )(xp, w_s, b_s, m_s)

    return out[:, :, :T] if padded else out


# speedup vs baseline: 1.7695x; 1.0040x over previous
import jax
import jax.numpy as jnp
from jax.experimental import pallas as pl
from jax.experimental.pallas import tpu as pltpu

_EPS = 1e-5
_VMEM_LIMIT = 64 * 1024 * 1024


def _affine_relu(x, w, b):
    y = jnp.dot(w, x.astype(jnp.float32), preferred_element_type=jnp.float32)
    return jnp.maximum(y + b, 0.0)


def _make_stats_kernel(rows_per_step, lanes, t_valid):
    def stats_kernel(x_ref, w_ref, b_ref, sum_ref, sumsq_ref):
        @pl.when(pl.program_id(0) == 0)
        def _init():
            sum_ref[...] = jnp.zeros_like(sum_ref)
            sumsq_ref[...] = jnp.zeros_like(sumsq_ref)

        wb = w_ref[...].astype(jnp.bfloat16)
        b = b_ref[...]
        h = wb.shape[0]
        t = x_ref.shape[-1]
        if t_valid != t:
            col = jax.lax.broadcasted_iota(jnp.int32, (1, t), 1)
            valid = col < t_valid
        s = jnp.zeros((h, lanes), jnp.float32)
        s2 = jnp.zeros((h, lanes), jnp.float32)
        for i in range(rows_per_step):
            xb = x_ref[i].astype(jnp.bfloat16)
            a = jnp.dot(wb, xb, preferred_element_type=jnp.float32)
            y = jnp.maximum(a + b, 0.0)
            if t_valid != t:
                y = jnp.where(valid, y, 0.0)
            for j in range(t // lanes):
                yj = y[:, j * lanes:(j + 1) * lanes]
                s = s + yj
                s2 = s2 + yj * yj
        sum_ref[...] += s
        sumsq_ref[...] += s2

    return stats_kernel


def _make_norm_kernel(rows_per_step):
    def norm_kernel(x_ref, w_ref, b_ref, m_ref, o_ref):
        w = w_ref[...]
        b = b_ref[...]
        m = m_ref[...]
        for i in range(rows_per_step):
            y = _affine_relu(x_ref[i], w, b)
            o_ref[i] = (y - m).astype(o_ref.dtype)

    return norm_kernel


def kernel(x, weight, bias):
    N, Cin, T = x.shape
    H = weight.shape[0]
    out_dtype = x.dtype
    lanes = 128

    T_pad = (T + lanes - 1) // lanes * lanes
    padded = T_pad != T
    xp = jnp.pad(x, ((0, 0), (0, 0), (0, T_pad - T))) if padded else x

    w2d = weight[:, :, 0].astype(jnp.float32)
    b2d = bias.reshape(H, 1).astype(jnp.float32)

    R = 16
    while N % R:
        R //= 2

    stats_x_spec = pl.BlockSpec((R, Cin, T_pad), lambda n: (n, 0, 0))
    w_spec = pl.BlockSpec((H, Cin), lambda n: (0, 0))
    col_spec = pl.BlockSpec((H, 1), lambda n: (0, 0))
    acc_spec = pl.BlockSpec((1, H, lanes), lambda n: (0, 0, 0))

    sums, sumsqs = pl.pallas_call(
        _make_stats_kernel(R, lanes, T),
        out_shape=(jax.ShapeDtypeStruct((1, H, lanes), jnp.float32),
                   jax.ShapeDtypeStruct((1, H, lanes), jnp.float32)),
        grid=(N // R,),
        in_specs=[stats_x_spec, w_spec, col_spec],
        out_specs=(acc_spec, acc_spec),
        compiler_params=pltpu.CompilerParams(
            dimension_semantics=("arbitrary",),
            vmem_limit_bytes=_VMEM_LIMIT),
    )(xp, w2d, b2d)

    count = float(N) * float(T)
    sum_c = jnp.sum(sums, axis=(0, 2)).reshape(H, 1)
    sumsq_c = jnp.sum(sumsqs, axis=(0, 2)).reshape(H, 1)
    mean = sum_c / count
    var = jnp.maximum(sumsq_c / count - mean * mean, 0.0)
    inv = jax.lax.rsqrt(var + _EPS)
    w_s = w2d * inv
    b_s = b2d * inv
    m_s = mean * inv

    Nb = 4
    while N % Nb:
        Nb //= 2
    norm_x_spec = pl.BlockSpec((Nb, Cin, T_pad), lambda n: (n, 0, 0))
    norm_w_spec = pl.BlockSpec((H, Cin), lambda n: (0, 0))
    norm_col_spec = pl.BlockSpec((H, 1), lambda n: (0, 0))
    out_spec = pl.BlockSpec((Nb, H, T_pad), lambda n: (n, 0, 0))

    out = pl.pallas_call(
        _make_norm_kernel(Nb),
        out_shape=jax.ShapeDtypeStruct((N, H, T_pad), out_dtype),
        grid=(N // Nb,),
        in_specs=[norm_x_spec, norm_w_spec, norm_col_spec, norm_col_spec],
        out_specs=out_spec,
        compiler_params=pltpu.CompilerParams(
            dimension_semantics=("arbitrary",),
            vmem_limit_bytes=_VMEM_LIMIT),
    )(xp, w_s, b_s, m_s)

    return out[:, :, :T] if padded else out


# speedup vs baseline: 1.7903x; 1.0117x over previous
import jax
import jax.numpy as jnp
from jax.experimental import pallas as pl
from jax.experimental.pallas import tpu as pltpu

_EPS = 1e-5
_VMEM_LIMIT = 64 * 1024 * 1024


def _affine_relu(x, w, b):
    y = jnp.dot(w, x.astype(jnp.float32), preferred_element_type=jnp.float32)
    return jnp.maximum(y + b, 0.0)


def _make_stats_kernel(rows_per_step, lanes, t_valid):
    def stats_kernel(x_ref, w_ref, b_ref, sum_ref, sumsq_ref):
        @pl.when(pl.program_id(0) == 0)
        def _init():
            sum_ref[...] = jnp.zeros_like(sum_ref)
            sumsq_ref[...] = jnp.zeros_like(sumsq_ref)

        wb = w_ref[...].astype(jnp.bfloat16)
        b = b_ref[...]
        h = wb.shape[0]
        t = x_ref.shape[-1]
        if t_valid != t:
            col = jax.lax.broadcasted_iota(jnp.int32, (1, t), 1)
            valid = col < t_valid
        s = jnp.zeros((h, lanes), jnp.float32)
        s2 = jnp.zeros((h, lanes), jnp.float32)
        for i in range(rows_per_step):
            xb = x_ref[i].astype(jnp.bfloat16)
            a = jnp.dot(wb, xb, preferred_element_type=jnp.float32)
            y = jnp.maximum(a + b, 0.0)
            if t_valid != t:
                y = jnp.where(valid, y, 0.0)
            for j in range(t // lanes):
                yj = y[:, j * lanes:(j + 1) * lanes]
                s = s + yj
                s2 = s2 + yj * yj
        sum_ref[...] += s
        sumsq_ref[...] += s2

    return stats_kernel


def _make_norm_kernel(rows_per_step):
    def norm_kernel(x_ref, w_ref, b_ref, m_ref, o_ref):
        w = w_ref[...]
        b = b_ref[...]
        m = m_ref[...]
        for i in range(rows_per_step):
            y = _affine_relu(x_ref[i], w, b)
            o_ref[i] = (y - m).astype(o_ref.dtype)

    return norm_kernel


def kernel(x, weight, bias):
    N, Cin, T = x.shape
    H = weight.shape[0]
    out_dtype = x.dtype
    lanes = 128

    T_pad = (T + lanes - 1) // lanes * lanes
    padded = T_pad != T
    xp = jnp.pad(x, ((0, 0), (0, 0), (0, T_pad - T))) if padded else x

    w2d = weight[:, :, 0].astype(jnp.float32)
    b2d = bias.reshape(H, 1).astype(jnp.float32)

    R = 32
    while N % R:
        R //= 2

    stats_x_spec = pl.BlockSpec((R, Cin, T_pad), lambda n: (n, 0, 0))
    w_spec = pl.BlockSpec((H, Cin), lambda n: (0, 0))
    col_spec = pl.BlockSpec((H, 1), lambda n: (0, 0))
    acc_spec = pl.BlockSpec((1, H, lanes), lambda n: (0, 0, 0))

    sums, sumsqs = pl.pallas_call(
        _make_stats_kernel(R, lanes, T),
        out_shape=(jax.ShapeDtypeStruct((1, H, lanes), jnp.float32),
                   jax.ShapeDtypeStruct((1, H, lanes), jnp.float32)),
        grid=(N // R,),
        in_specs=[stats_x_spec, w_spec, col_spec],
        out_specs=(acc_spec, acc_spec),
        compiler_params=pltpu.CompilerParams(
            dimension_semantics=("arbitrary",),
            vmem_limit_bytes=_VMEM_LIMIT),
    )(xp, w2d, b2d)

    count = float(N) * float(T)
    sum_c = jnp.sum(sums, axis=(0, 2)).reshape(H, 1)
    sumsq_c = jnp.sum(sumsqs, axis=(0, 2)).reshape(H, 1)
    mean = sum_c / count
    var = jnp.maximum(sumsq_c / count - mean * mean, 0.0)
    inv = jax.lax.rsqrt(var + _EPS)
    w_s = w2d * inv
    b_s = b2d * inv
    m_s = mean * inv

    Nb = 8
    while N % Nb:
        Nb //= 2
    norm_x_spec = pl.BlockSpec((Nb, Cin, T_pad), lambda n: (n, 0, 0))
    norm_w_spec = pl.BlockSpec((H, Cin), lambda n: (0, 0))
    norm_col_spec = pl.BlockSpec((H, 1), lambda n: (0, 0))
    out_spec = pl.BlockSpec((Nb, H, T_pad), lambda n: (n, 0, 0))

    out = pl.pallas_call(
        _make_norm_kernel(Nb),
        out_shape=jax.ShapeDtypeStruct((N, H, T_pad), out_dtype),
        grid=(N // Nb,),
        in_specs=[norm_x_spec, norm_w_spec, norm_col_spec, norm_col_spec],
        out_specs=out_spec,
        compiler_params=pltpu.CompilerParams(
            dimension_semantics=("arbitrary",),
            vmem_limit_bytes=_VMEM_LIMIT),
    )(xp, w_s, b_s, m_s)

    return out[:, :, :T] if padded else out
